# baseline (speedup 1.0000x reference)
import functools
import jax
import jax.numpy as jnp
from jax import lax
from jax.experimental import pallas as pl
from jax.experimental.pallas import tpu as pltpu
from jax.experimental.pallas import tpu_sc as plsc

B = 1024
N_FIELDS = 26
EMB_DIM = 32
D_CAT = 224
TRIU = D_CAT * (D_CAT + 1) // 2
CHUNK = 512
N_CHUNKS = 49
STREAM_END = N_CHUNKS * CHUNK
PACK = 128 // EMB_DIM
ROWS_PACKED = N_FIELDS * 128

N_IDX = B * N_FIELDS
IDX_CHUNK = 104
N_WORKERS = 32
IDX_PER_W = N_IDX // N_WORKERS
CHUNKS_PER_W = IDX_PER_W // IDX_CHUNK
IDX_ROWS = N_IDX // IDX_CHUNK


def _row_off(i):
    return D_CAT * i - (i * (i - 1)) // 2


def _segments(k0, k1):
    segs = []
    for i in range(D_CAT):
        lo = max(_row_off(i), k0)
        hi = min(_row_off(i) + D_CAT - i, k1)
        if lo < hi:
            segs.append((i, i + (lo - _row_off(i)), i + (hi - _row_off(i))))
    return segs


def _sc_gather(table128, idx2d):
    mesh = plsc.VectorSubcoreMesh(core_axis_name="c", subcore_axis_name="s")

    @functools.partial(
        pl.kernel,
        mesh=mesh,
        out_type=jax.ShapeDtypeStruct((N_IDX, 128), jnp.float32),
        scratch_types=[
            pltpu.VMEM((CHUNKS_PER_W, IDX_CHUNK), jnp.int32),
            pltpu.VMEM((IDX_PER_W, 128), jnp.float32),
            pltpu.SemaphoreType.DMA,
        ],
    )
    def k(table_hbm, idx_hbm, out_hbm, idx_v, rows_v, sem):
        info = plsc.get_sparse_core_info()
        nc = info.num_cores
        wid = lax.axis_index("s") * nc + lax.axis_index("c")
        pltpu.sync_copy(idx_hbm.at[pl.ds(wid * CHUNKS_PER_W, CHUNKS_PER_W)],
                        idx_v)
        copies = []
        for j in range(CHUNKS_PER_W):
            copies.append(pltpu.make_async_copy(
                table_hbm.at[idx_v.at[j]],
                rows_v.at[pl.ds(j * IDX_CHUNK, IDX_CHUNK)],
                sem))
        for c in copies:
            c.start()
        for c in copies:
            c.wait()
        pltpu.sync_copy(rows_v, out_hbm.at[pl.ds(wid * IDX_PER_W, IDX_PER_W)])

    return k(table128, idx2d)


def _dot_t(x, w):
    return lax.dot_general(x, w, (((1,), (1,)), ((), ())),
                           preferred_element_type=jnp.float32)


def _tc_body(rows_ref, xmod_ref, xe_ref, xd_ref,
             pj_w1_ref, pj_b1_ref, pj_w2_ref, pj_b2_ref, pj_g_ref, pj_bt_ref,
             bm_w1_ref, bm_b1_ref, bm_w2_ref, bm_b2_ref,
             w1_ref, w1tail_ref, tm_b1_ref, tm_w2_ref, tm_b2_ref,
             tm_w3_ref, tm_b3_ref,
             out_ref, xemb_ref,
             c_s, bot_s, acc_ref):
    g = pl.program_id(0)

    @pl.when(g == 0)
    def _prologue():
        h = jnp.maximum(_dot_t(xe_ref[...], pj_w1_ref[...]) + pj_b1_ref[...],
                        0.0)
        h = _dot_t(h, pj_w2_ref[...]) + pj_b2_ref[...]
        mean = jnp.mean(h, axis=0, keepdims=True)
        var = jnp.mean((h - mean) * (h - mean), axis=0, keepdims=True)
        x_embed = (pj_g_ref[...] * (h - mean) * lax.rsqrt(var + 1e-5)
                   + pj_bt_ref[...])
        xemb_ref[...] = x_embed

        bot = jnp.maximum(_dot_t(xd_ref[...], bm_w1_ref[...])
                          + bm_b1_ref[...], 0.0)
        bot = jnp.maximum(_dot_t(bot, bm_w2_ref[...]) + bm_b2_ref[...], 0.0)
        bot_s[...] = bot

        f_of_l = jax.lax.broadcasted_iota(jnp.int32, (N_FIELDS, ROWS_PACKED),
                                          1) // 128
        f_id = jax.lax.broadcasted_iota(jnp.int32, (N_FIELDS, ROWS_PACKED), 0)
        f_onehot = (f_of_l == f_id).astype(jnp.float32)
        xmod_b = lax.dot_general(xmod_ref[...], f_onehot,
                                 (((1,), (0,)), ((), ())),
                                 preferred_element_type=jnp.float32)
        l_iota = jax.lax.broadcasted_iota(jnp.int32, (1, ROWS_PACKED), 1)
        qpat = ((l_iota // EMB_DIM) % PACK).astype(jnp.float32)
        mexp = (xmod_b == qpat).astype(jnp.float32)
        l_mod = jax.lax.broadcasted_iota(jnp.int32, (ROWS_PACKED, EMB_DIM),
                                         0) % EMB_DIM
        d_id = jax.lax.broadcasted_iota(jnp.int32, (ROWS_PACKED, EMB_DIM), 1)
        sel = (l_mod == d_id).astype(jnp.float32)
        embed_x = lax.dot_general(rows_ref[...] * mexp, sel,
                                  (((1,), (0,)), ((), ())),
                                  preferred_element_type=jnp.float32)

        c = jnp.concatenate([bot, x_embed, embed_x], axis=1)
        c_s[...] = c

        segs = [c[:, i:i + 1] * c[:, jl:jh]
                for i, jl, jh in _segments(STREAM_END, TRIU)]
        segs.append(bot)
        tail_prod = jnp.concatenate(segs, axis=1)
        acc_ref[...] = _dot_t(tail_prod, w1tail_ref[...]) + tm_b1_ref[...]

    for k in range(N_CHUNKS):
        @pl.when(g == k)
        def _chunk(k=k):
            c = c_s[...]
            segs = [c[:, i:i + 1] * c[:, jl:jh]
                    for i, jl, jh in _segments(k * CHUNK, (k + 1) * CHUNK)]
            prod = jnp.concatenate(segs, axis=1)
            acc_ref[...] = acc_ref[...] + _dot_t(prod, w1_ref[...])

    @pl.when(g == N_CHUNKS - 1)
    def _epilogue():
        t = jnp.maximum(acc_ref[...], 0.0)
        t = jnp.maximum(_dot_t(t, tm_w2_ref[...]) + tm_b2_ref[...], 0.0)
        logit = _dot_t(t, tm_w3_ref[...])[:, 0:1] + tm_b3_ref[0, 0]
        out_ref[...] = jax.nn.sigmoid(logit)


def kernel(x_sparse, x_dense, x_embed_before_projection, emb_table,
           pj_w1, pj_b1, pj_w2, pj_b2, pj_gamma, pj_beta,
           bm_w1, bm_b1, bm_w2, bm_b2,
           tm_w1, tm_b1, tm_w2, tm_b2, tm_w3, tm_b3):
    xs = x_sparse.astype(jnp.int32)
    idx2d = (xs // PACK).reshape(IDX_ROWS, IDX_CHUNK)
    xmod = (xs % PACK).astype(jnp.float32)
    table128 = emb_table.reshape(1000000 // PACK, 128)
    rows = _sc_gather(table128, idx2d)
    rows3328 = rows.reshape(B, ROWS_PACKED)

    def full(shape):
        nd = len(shape)
        return pl.BlockSpec(shape, lambda g, _nd=nd: (0,) * _nd)

    in_specs = [
        full((B, ROWS_PACKED)), full((B, N_FIELDS)),
        full((B, 512)), full((B, 256)),
        full((256, 512)), full((1, 256)), full((64, 256)), full((1, 64)),
        full((1, 64)), full((1, 64)),
        full((256, 256)), full((1, 256)), full((128, 256)), full((1, 128)),
        pl.BlockSpec((512, CHUNK), lambda g: (0, g)),
        full((512, 240)), full((1, 512)), full((256, 512)), full((1, 256)),
        full((8, 256)),
        pl.BlockSpec(memory_space=pltpu.SMEM),
    ]

    out, xemb = pl.pallas_call(
        _tc_body,
        grid=(N_CHUNKS,),
        out_shape=(jax.ShapeDtypeStruct((B, 1), jnp.float32),
                   jax.ShapeDtypeStruct((B, 64), jnp.float32)),
        in_specs=in_specs,
        out_specs=(pl.BlockSpec((B, 1), lambda g: (0, 0)),
                   pl.BlockSpec((B, 64), lambda g: (0, 0))),
        scratch_shapes=[
            pltpu.VMEM((B, D_CAT), jnp.float32),
            pltpu.VMEM((B, 128), jnp.float32),
            pltpu.VMEM((B, 512), jnp.float32),
        ],
    )(rows3328, xmod, x_embed_before_projection, x_dense,
      pj_w1, pj_b1.reshape(1, -1), pj_w2, pj_b2.reshape(1, -1),
      pj_gamma.reshape(1, -1), pj_beta.reshape(1, -1),
      bm_w1, bm_b1.reshape(1, -1), bm_w2, bm_b2.reshape(1, -1),
      tm_w1, tm_w1[:, STREAM_END:], tm_b1.reshape(1, -1),
      tm_w2, tm_b2.reshape(1, -1), jnp.pad(tm_w3, ((0, 7), (0, 0))),
      tm_b3.reshape(1, 1))
    return (out, xemb)


# speedup vs baseline: 17.8642x; 17.8642x over previous
import jax
import jax.numpy as jnp
import numpy as np
from jax import lax
from jax.experimental import pallas as pl
from jax.experimental.pallas import tpu as pltpu

B = 1024
N_FIELDS = 26
EMB_DIM = 32
D_CAT = 224
TRIU = D_CAT * (D_CAT + 1) // 2
CHUNK = 512
N_CHUNKS = 49
STREAM_END = N_CHUNKS * CHUNK


def _row_off(i):
    return D_CAT * i - (i * (i - 1)) // 2


def _segments(k0, k1):
    segs = []
    for i in range(D_CAT):
        lo = max(_row_off(i), k0)
        hi = min(_row_off(i) + D_CAT - i, k1)
        if lo < hi:
            segs.append((i, i + (lo - _row_off(i)), i + (hi - _row_off(i))))
    return segs


def _select_mats():
    su = np.zeros((STREAM_END, D_CAT), np.float32)
    sv = np.zeros((STREAM_END, D_CAT), np.float32)
    k = 0
    for i in range(D_CAT):
        w = D_CAT - i
        for j in range(i, D_CAT):
            if k >= STREAM_END:
                break
            su[k, i] = 1.0
            sv[k, j] = 1.0
            k += 1
    return su, sv


_SU, _SV = _select_mats()


def _dot_t(x, w):
    return lax.dot_general(x, w, (((1,), (1,)), ((), ())),
                           preferred_element_type=jnp.float32)


def _prologue_body(rows_ref, xe_ref, xd_ref,
                   pj_w1_ref, pj_b1_ref, pj_w2_ref, pj_b2_ref,
                   pj_g_ref, pj_bt_ref,
                   bm_w1_ref, bm_b1_ref, bm_w2_ref, bm_b2_ref,
                   w1tail_ref, tm_b1_ref,
                   xemb_ref, c_ref, acc0_ref):
    h = jnp.maximum(_dot_t(xe_ref[...], pj_w1_ref[...]) + pj_b1_ref[...], 0.0)
    h = _dot_t(h, pj_w2_ref[...]) + pj_b2_ref[...]
    mean = jnp.mean(h, axis=0, keepdims=True)
    var = jnp.mean((h - mean) * (h - mean), axis=0, keepdims=True)
    x_embed = (pj_g_ref[...] * (h - mean) * lax.rsqrt(var + 1e-5)
               + pj_bt_ref[...])
    xemb_ref[...] = x_embed

    bot = jnp.maximum(_dot_t(xd_ref[...], bm_w1_ref[...]) + bm_b1_ref[...],
                      0.0)
    bot = jnp.maximum(_dot_t(bot, bm_w2_ref[...]) + bm_b2_ref[...], 0.0)

    r_mod = jax.lax.broadcasted_iota(jnp.int32,
                                     (N_FIELDS * EMB_DIM, EMB_DIM), 0)
    c_id = jax.lax.broadcasted_iota(jnp.int32,
                                    (N_FIELDS * EMB_DIM, EMB_DIM), 1)
    sel = (r_mod % EMB_DIM == c_id).astype(jnp.float32)
    embed_x = lax.dot_general(rows_ref[...], sel, (((1,), (0,)), ((), ())),
                              preferred_element_type=jnp.float32)

    c = jnp.concatenate([bot, x_embed, embed_x], axis=1)
    c_ref[...] = c

    segs = [c[:, i:i + 1] * c[:, jl:jh]
            for i, jl, jh in _segments(STREAM_END, TRIU)]
    segs.append(bot)
    tail_prod = jnp.concatenate(segs, axis=1)
    acc0_ref[...] = _dot_t(tail_prod, w1tail_ref[...]) + tm_b1_ref[...]


def _interact_body(c_ref, su_ref, sv_ref, w1_ref, acc_ref):
    g = pl.program_id(0)
    c = c_ref[...]
    cu = _dot_t(c, su_ref[...])
    cv = _dot_t(c, sv_ref[...])
    d = _dot_t(cu * cv, w1_ref[...])

    @pl.when(g == 0)
    def _init():
        acc_ref[...] = d

    @pl.when(g != 0)
    def _accum():
        acc_ref[...] = acc_ref[...] + d


def _epilogue_body(acc_ref, acc0_ref, tm_w2_ref, tm_b2_ref,
                   tm_w3_ref, tm_b3_ref, out_ref):
    t = jnp.maximum(acc_ref[...] + acc0_ref[...], 0.0)
    t = jnp.maximum(_dot_t(t, tm_w2_ref[...]) + tm_b2_ref[...], 0.0)
    logit = _dot_t(t, tm_w3_ref[...])[:, 0:1] + tm_b3_ref[0, 0]
    out_ref[...] = jax.nn.sigmoid(logit)


def kernel(x_sparse, x_dense, x_embed_before_projection, emb_table,
           pj_w1, pj_b1, pj_w2, pj_b2, pj_gamma, pj_beta,
           bm_w1, bm_b1, bm_w2, bm_b2,
           tm_w1, tm_b1, tm_w2, tm_b2, tm_w3, tm_b3):
    rows = jnp.take(emb_table, x_sparse.astype(jnp.int32).reshape(-1),
                    axis=0)
    rows832 = rows.reshape(B, N_FIELDS * EMB_DIM)

    vmem = pl.BlockSpec(memory_space=pltpu.VMEM)

    xemb, c, acc0 = pl.pallas_call(
        _prologue_body,
        out_shape=(jax.ShapeDtypeStruct((B, 64), jnp.float32),
                   jax.ShapeDtypeStruct((B, D_CAT), jnp.float32),
                   jax.ShapeDtypeStruct((B, 512), jnp.float32)),
        in_specs=[vmem] * 15,
        out_specs=(vmem, vmem, vmem),
    )(rows832, x_embed_before_projection, x_dense,
      pj_w1, pj_b1.reshape(1, -1), pj_w2, pj_b2.reshape(1, -1),
      pj_gamma.reshape(1, -1), pj_beta.reshape(1, -1),
      bm_w1, bm_b1.reshape(1, -1), bm_w2, bm_b2.reshape(1, -1),
      tm_w1[:, STREAM_END:], tm_b1.reshape(1, -1))

    def full(shape):
        nd = len(shape)
        return pl.BlockSpec(shape, lambda g, _nd=nd: (0,) * _nd)

    acc = pl.pallas_call(
        _interact_body,
        grid=(N_CHUNKS,),
        out_shape=jax.ShapeDtypeStruct((B, 512), jnp.float32),
        in_specs=[
            full((B, D_CAT)),
            pl.BlockSpec((CHUNK, D_CAT), lambda g: (g, 0)),
            pl.BlockSpec((CHUNK, D_CAT), lambda g: (g, 0)),
            pl.BlockSpec((512, CHUNK), lambda g: (0, g)),
        ],
        out_specs=full((B, 512)),
    )(c, jnp.asarray(_SU), jnp.asarray(_SV), tm_w1)

    out, = pl.pallas_call(
        _epilogue_body,
        out_shape=(jax.ShapeDtypeStruct((B, 1), jnp.float32),),
        in_specs=[vmem, vmem, vmem, vmem, vmem,
                  pl.BlockSpec(memory_space=pltpu.SMEM)],
        out_specs=(vmem,),
    )(acc, acc0, tm_w2, tm_b2.reshape(1, -1),
      jnp.pad(tm_w3, ((0, 7), (0, 0))), tm_b3.reshape(1, 1))
    return (out, xemb)
